# baseline (speedup 1.0000x reference)
import functools

import jax
import jax.numpy as jnp
from jax.experimental import pallas as pl
from jax.experimental.pallas import tpu as pltpu

EPS = 1e-8
NUM_CLASSES = 1000
TOP_K = 16
BLK = 512
NEG = -jnp.inf
BIG_I32 = 2**31 - 1


def _knn_kernel(q_ref, k_ref, db_ref, unit_ref, cls_ref, avg_ref, topv_ref,
                qn_s, run_v, run_c, b_v, votes, *, nblocks, kvalid, q):
    j = pl.program_id(0)

    @pl.when(j == 0)
    def _init():
        qv = q_ref[...]
        qn = jnp.sqrt(jnp.sum(qv * qv, axis=1, keepdims=True))
        qn_s[...] = qv / (qn + EPS)
        run_v[...] = jnp.full((q, TOP_K), NEG, jnp.float32)
        run_c[...] = jnp.zeros((q, TOP_K), jnp.int32)

    kb = k_ref[...]
    kn = jnp.sqrt(jnp.sum(kb * kb, axis=1, keepdims=True))
    kb = kb / (kn + EPS)
    sim = jax.lax.dot_general(qn_s[...], kb, (((1,), (1,)), ((), ())),
                              preferred_element_type=jnp.float32)
    col = j * BLK + jax.lax.broadcasted_iota(jnp.int32, (q, BLK), 1)
    sim = jnp.where(col < kvalid, sim, NEG)
    b_v[...] = sim

    thresh = run_v[:, TOP_K - 1:TOP_K]
    cnt = jnp.sum((sim > thresh).astype(jnp.int32), axis=1, keepdims=True)
    needed = jnp.max(cnt)

    lane = jax.lax.broadcasted_iota(jnp.int32, (q, BLK), 1)
    li = jax.lax.broadcasted_iota(jnp.int32, (q, TOP_K), 1)
    dbrow = db_ref[0, 0, :][None, :]
    for t in range(TOP_K):
        @pl.when(t < needed)
        def _round():
            v = b_v[...]
            m = jnp.max(v, axis=1, keepdims=True)
            pos = jnp.where(v == m, lane, BIG_I32)
            psel = jnp.min(pos, axis=1, keepdims=True)
            hit = lane == psel
            c = jnp.max(jnp.where(hit, dbrow, -1), axis=1, keepdims=True)
            b_v[...] = jnp.where(hit, NEG, v)
            rv = run_v[...]
            rc = run_c[...]
            do = m > rv[:, TOP_K - 1:TOP_K]
            ipos = jnp.sum((rv >= m).astype(jnp.int32), axis=1, keepdims=True)
            sh_v = jnp.roll(rv, 1, axis=1)
            sh_c = jnp.roll(rc, 1, axis=1)
            nv = jnp.where(li < ipos, rv, jnp.where(li == ipos, m, sh_v))
            nc = jnp.where(li < ipos, rc, jnp.where(li == ipos, c, sh_c))
            run_v[...] = jnp.where(do, nv, rv)
            run_c[...] = jnp.where(do, nc, rc)

    @pl.when(j == nblocks - 1)
    def _fin():
        tv = run_v[...]
        tc = run_c[...]
        unit_ref[...] = tv[:, 0:1]
        avg_ref[...] = jnp.mean(tv, axis=1, keepdims=True)
        topv_ref[...] = tv
        ci = jax.lax.broadcasted_iota(jnp.int32, (q, 1024), 1)
        acc = jnp.where(ci < NUM_CLASSES, 0.0, -1e30)
        for t in range(TOP_K):
            acc = acc + jnp.where(ci == tc[:, t:t + 1], tv[:, t:t + 1], 0.0)
        votes[...] = acc
        vm = jnp.max(acc, axis=1, keepdims=True)
        cpos = jnp.where(acc == vm, ci, BIG_I32)
        cls_ref[...] = jnp.min(cpos, axis=1, keepdims=True)


def kernel(queries, keys, db_classes, k):
    del k
    q, d = queries.shape
    kvalid = keys.shape[0]
    nblocks = (kvalid + BLK - 1) // BLK
    kpad = nblocks * BLK
    keys_p = jnp.pad(keys, ((0, kpad - kvalid), (0, 0)))
    db_p = jnp.pad(db_classes, (0, kpad - kvalid)).reshape(nblocks, 1, BLK)

    out = pl.pallas_call(
        functools.partial(_knn_kernel, nblocks=nblocks, kvalid=kvalid, q=q),
        grid=(nblocks,),
        in_specs=[
            pl.BlockSpec((q, d), lambda j: (0, 0)),
            pl.BlockSpec((BLK, d), lambda j: (j, 0)),
            pl.BlockSpec((1, 1, BLK), lambda j: (j, 0, 0)),
        ],
        out_specs=[
            pl.BlockSpec((q, 1), lambda j: (0, 0)),
            pl.BlockSpec((q, 1), lambda j: (0, 0)),
            pl.BlockSpec((q, 1), lambda j: (0, 0)),
            pl.BlockSpec((q, TOP_K), lambda j: (0, 0)),
        ],
        out_shape=[
            jax.ShapeDtypeStruct((q, 1), jnp.float32),
            jax.ShapeDtypeStruct((q, 1), jnp.int32),
            jax.ShapeDtypeStruct((q, 1), jnp.float32),
            jax.ShapeDtypeStruct((q, TOP_K), jnp.float32),
        ],
        scratch_shapes=[
            pltpu.VMEM((q, d), jnp.float32),
            pltpu.VMEM((q, TOP_K), jnp.float32),
            pltpu.VMEM((q, TOP_K), jnp.int32),
            pltpu.VMEM((q, BLK), jnp.float32),
            pltpu.VMEM((q, 1024), jnp.float32),
        ],
        compiler_params=pltpu.CompilerParams(
            dimension_semantics=("arbitrary",)),
    )(queries, keys_p, db_p)
    unit, cls_, avg, topv = out
    return (unit[:, 0], cls_[:, 0], avg[:, 0], topv)


# speedup vs baseline: 5.7982x; 5.7982x over previous
import functools

import jax
import jax.numpy as jnp
from jax.experimental import pallas as pl
from jax.experimental.pallas import tpu as pltpu

EPS = 1e-8
NUM_CLASSES = 1000
TOP_K = 16
BLK = 512
NG = 128
NEG = -jnp.inf
BIG_I32 = 2**31 - 1


def _lex_ge(av, ap, bv, bp):
    return (av > bv) | ((av == bv) & (ap < bp))


def _knn_kernel(q_ref, k_ref, db_ref, stats_ref, cls_ref, topv_ref,
                qn_s, run_v, run_p, stk_v, stk_p, *, nblocks, kvalid, q):
    j = pl.program_id(0)

    @pl.when(j == 0)
    def _init():
        qv = q_ref[...]
        qn = jnp.sqrt(jnp.sum(qv * qv, axis=1, keepdims=True))
        qn_s[...] = qv / (qn + EPS)
        run_v[...] = jnp.full((TOP_K, q), NEG, jnp.float32)
        run_p[...] = jnp.full((TOP_K, q), BIG_I32, jnp.int32)

    kb = k_ref[...]
    kn = jnp.sqrt(jnp.sum(kb * kb, axis=1, keepdims=True))
    kb = kb / (kn + EPS)
    sim = jax.lax.dot_general(kb, qn_s[...], (((1,), (1,)), ((), ())),
                              preferred_element_type=jnp.float32)
    rowi = jax.lax.broadcasted_iota(jnp.int32, (BLK, q), 0)
    sim = jnp.where(j * BLK + rowi < kvalid, sim, NEG)
    pay = (j * BLK + rowi) * 1024 + db_ref[0][:, 0:1]

    thresh = run_v[TOP_K - 1:TOP_K, :]
    cnt = jnp.sum((sim > thresh).astype(jnp.int32), axis=0, keepdims=True)
    needed = jnp.max(cnt)

    c = [(sim[s * NG:(s + 1) * NG, :], pay[s * NG:(s + 1) * NG, :])
         for s in range(4)]
    for a, b in ((0, 1), (2, 3), (0, 2), (1, 3), (1, 2)):
        keep = _lex_ge(c[a][0], c[a][1], c[b][0], c[b][1])
        hi = (jnp.where(keep, c[a][0], c[b][0]), jnp.where(keep, c[a][1], c[b][1]))
        lo = (jnp.where(keep, c[b][0], c[a][0]), jnp.where(keep, c[b][1], c[a][1]))
        c[a], c[b] = hi, lo
    for s in range(4):
        stk_v[s * NG:(s + 1) * NG, :] = c[s][0]
        stk_p[s * NG:(s + 1) * NG, :] = c[s][1]

    li = jax.lax.broadcasted_iota(jnp.int32, (TOP_K, q), 0)

    def _round():
        top_v = stk_v[0:NG, :]
        top_p = stk_p[0:NG, :]
        m = jnp.max(top_v, axis=0, keepdims=True)
        candp = jnp.where(top_v == m, top_p, BIG_I32)
        psel = jnp.min(candp, axis=0, keepdims=True)
        hit = top_p == psel
        for s in range(3):
            lo_v = stk_v[(s + 1) * NG:(s + 2) * NG, :]
            lo_p = stk_p[(s + 1) * NG:(s + 2) * NG, :]
            cu_v = stk_v[s * NG:(s + 1) * NG, :]
            cu_p = stk_p[s * NG:(s + 1) * NG, :]
            stk_v[s * NG:(s + 1) * NG, :] = jnp.where(hit, lo_v, cu_v)
            stk_p[s * NG:(s + 1) * NG, :] = jnp.where(hit, lo_p, cu_p)
        stk_v[3 * NG:4 * NG, :] = jnp.where(hit, NEG, stk_v[3 * NG:4 * NG, :])
        stk_p[3 * NG:4 * NG, :] = jnp.where(hit, BIG_I32, stk_p[3 * NG:4 * NG, :])
        rv = run_v[...]
        rp = run_p[...]
        do = m > rv[TOP_K - 1:TOP_K, :]
        ipos = jnp.sum(_lex_ge(rv, rp, m, psel).astype(jnp.int32),
                       axis=0, keepdims=True)
        sh_v = jnp.roll(rv, 1, axis=0)
        sh_p = jnp.roll(rp, 1, axis=0)
        nv = jnp.where(li < ipos, rv, jnp.where(li == ipos, m, sh_v))
        np_ = jnp.where(li < ipos, rp, jnp.where(li == ipos, psel, sh_p))
        run_v[...] = jnp.where(do, nv, rv)
        run_p[...] = jnp.where(do, np_, rp)

    for g in range(TOP_K // 4):
        @pl.when(g * 4 < needed)
        def _grp():
            for _ in range(4):
                _round()

    @pl.when(j == nblocks - 1)
    def _fin():
        tv = run_v[...]
        cls16 = jnp.bitwise_and(run_p[...], 1023)
        unit = tv[0:1, :]
        avg = jnp.mean(tv, axis=0, keepdims=True)
        topv_ref[...] = tv
        stats_ref[...] = jnp.concatenate(
            [unit, avg, jnp.zeros((6, q), jnp.float32)], axis=0)
        ci = jax.lax.broadcasted_iota(jnp.int32, (1024, q), 0)
        acc = jnp.where(ci < NUM_CLASSES, 0.0, -1e30)
        for t in range(TOP_K):
            acc = acc + jnp.where(ci == cls16[t:t + 1, :], tv[t:t + 1, :], 0.0)
        vm = jnp.max(acc, axis=0, keepdims=True)
        cpos = jnp.where(acc == vm, ci, BIG_I32)
        cls_ref[...] = jnp.concatenate(
            [jnp.min(cpos, axis=0, keepdims=True),
             jnp.zeros((7, q), jnp.int32)], axis=0)


def kernel(queries, keys, db_classes, k):
    del k
    q, d = queries.shape
    kvalid = keys.shape[0]
    nblocks = (kvalid + BLK - 1) // BLK
    kpad = nblocks * BLK
    keys_p = jnp.pad(keys, ((0, kpad - kvalid), (0, 0)))
    db_p = jnp.broadcast_to(
        jnp.pad(db_classes, (0, kpad - kvalid))[:, None],
        (kpad, 128)).reshape(nblocks, BLK, 128)

    out = pl.pallas_call(
        functools.partial(_knn_kernel, nblocks=nblocks, kvalid=kvalid, q=q),
        grid=(nblocks,),
        in_specs=[
            pl.BlockSpec((q, d), lambda j: (0, 0)),
            pl.BlockSpec((BLK, d), lambda j: (j, 0)),
            pl.BlockSpec((1, BLK, 128), lambda j: (j, 0, 0)),
        ],
        out_specs=[
            pl.BlockSpec((8, q), lambda j: (0, 0)),
            pl.BlockSpec((8, q), lambda j: (0, 0)),
            pl.BlockSpec((TOP_K, q), lambda j: (0, 0)),
        ],
        out_shape=[
            jax.ShapeDtypeStruct((8, q), jnp.float32),
            jax.ShapeDtypeStruct((8, q), jnp.int32),
            jax.ShapeDtypeStruct((TOP_K, q), jnp.float32),
        ],
        scratch_shapes=[
            pltpu.VMEM((q, d), jnp.float32),
            pltpu.VMEM((TOP_K, q), jnp.float32),
            pltpu.VMEM((TOP_K, q), jnp.int32),
            pltpu.VMEM((BLK, q), jnp.float32),
            pltpu.VMEM((BLK, q), jnp.int32),
        ],
        compiler_params=pltpu.CompilerParams(
            dimension_semantics=("arbitrary",)),
    )(queries, keys_p, db_p)
    stats, clsout, topv_t = out
    return (stats[0], clsout[0], stats[1], jnp.transpose(topv_t))


# speedup vs baseline: 5.8922x; 1.0162x over previous
import functools

import jax
import jax.numpy as jnp
from jax.experimental import pallas as pl
from jax.experimental.pallas import tpu as pltpu

EPS = 1e-8
NUM_CLASSES = 1000
TOP_K = 16
BLK = 512
NG = 128
NEG = -jnp.inf
BIG_I32 = 2**31 - 1


def _lex_ge(av, ap, bv, bp):
    return (av > bv) | ((av == bv) & (ap < bp))


def _knn_kernel(q_ref, k_ref, pay_ref, stats_ref, cls_ref, topv_ref,
                qn_s, run_v, run_p, stk_v, stk_p, flag, *, nblocks, kvalid, q):
    j = pl.program_id(0)

    @pl.when(j == 0)
    def _init():
        qv = q_ref[...]
        qn = jnp.sqrt(jnp.sum(qv * qv, axis=1, keepdims=True))
        qn_s[...] = qv / (qn + EPS)
        run_v[...] = jnp.full((TOP_K, q), NEG, jnp.float32)
        run_p[...] = jnp.full((TOP_K, q), BIG_I32, jnp.int32)

    kb = k_ref[...]
    kn = jnp.sqrt(jnp.sum(kb * kb, axis=1, keepdims=True))
    kb = kb / (kn + EPS)
    sim = jax.lax.dot_general(kb, qn_s[...], (((1,), (1,)), ((), ())),
                              preferred_element_type=jnp.float32)
    rowi = jax.lax.broadcasted_iota(jnp.int32, (BLK, q), 0)
    sim = jnp.where(j * BLK + rowi < kvalid, sim, NEG)
    pay = pay_ref[0][:, 0:1]

    c = [(sim[s * NG:(s + 1) * NG, :], pay[s * NG:(s + 1) * NG, :])
         for s in range(4)]
    for a, b in ((0, 1), (2, 3), (0, 2), (1, 3), (1, 2)):
        keep = _lex_ge(c[a][0], c[a][1], c[b][0], c[b][1])
        hi = (jnp.where(keep, c[a][0], c[b][0]), jnp.where(keep, c[a][1], c[b][1]))
        lo = (jnp.where(keep, c[b][0], c[a][0]), jnp.where(keep, c[b][1], c[a][1]))
        c[a], c[b] = hi, lo
    for s in range(4):
        stk_v[s * NG:(s + 1) * NG, :] = c[s][0]
        stk_p[s * NG:(s + 1) * NG, :] = c[s][1]

    li = jax.lax.broadcasted_iota(jnp.int32, (TOP_K, q), 0)

    def _round():
        top_v = stk_v[0:NG, :]
        top_p = stk_p[0:NG, :]
        m = jnp.max(top_v, axis=0, keepdims=True)
        candp = jnp.where(top_v == m, top_p, BIG_I32)
        psel = jnp.min(candp, axis=0, keepdims=True)
        hit = top_p == psel
        for s in range(3):
            lo_v = stk_v[(s + 1) * NG:(s + 2) * NG, :]
            lo_p = stk_p[(s + 1) * NG:(s + 2) * NG, :]
            cu_v = stk_v[s * NG:(s + 1) * NG, :]
            cu_p = stk_p[s * NG:(s + 1) * NG, :]
            stk_v[s * NG:(s + 1) * NG, :] = jnp.where(hit, lo_v, cu_v)
            stk_p[s * NG:(s + 1) * NG, :] = jnp.where(hit, lo_p, cu_p)
        stk_v[3 * NG:4 * NG, :] = jnp.where(hit, NEG, stk_v[3 * NG:4 * NG, :])
        stk_p[3 * NG:4 * NG, :] = jnp.where(hit, BIG_I32, stk_p[3 * NG:4 * NG, :])
        rv = run_v[...]
        rp = run_p[...]
        do = m > rv[TOP_K - 1:TOP_K, :]
        ipos = jnp.sum(_lex_ge(rv, rp, m, psel).astype(jnp.int32),
                       axis=0, keepdims=True)
        sh_v = jnp.roll(rv, 1, axis=0)
        sh_p = jnp.roll(rp, 1, axis=0)
        nv = jnp.where(li < ipos, rv, jnp.where(li == ipos, m, sh_v))
        np_ = jnp.where(li < ipos, rp, jnp.where(li == ipos, psel, sh_p))
        run_v[...] = jnp.where(do, nv, rv)
        run_p[...] = jnp.where(do, np_, rp)
        flag[0] = jnp.max(do.astype(jnp.int32))

    _round()
    for _ in range(TOP_K - 1):
        @pl.when(flag[0] > 0)
        def _gated():
            _round()

    @pl.when(j == nblocks - 1)
    def _fin():
        tv = run_v[...]
        cls16 = jnp.bitwise_and(run_p[...], 1023)
        unit = tv[0:1, :]
        avg = jnp.mean(tv, axis=0, keepdims=True)
        topv_ref[...] = tv
        stats_ref[...] = jnp.concatenate(
            [unit, avg, jnp.zeros((6, q), jnp.float32)], axis=0)
        ci = jax.lax.broadcasted_iota(jnp.int32, (1024, q), 0)
        acc = jnp.where(ci < NUM_CLASSES, 0.0, -1e30)
        for t in range(TOP_K):
            acc = acc + jnp.where(ci == cls16[t:t + 1, :], tv[t:t + 1, :], 0.0)
        vm = jnp.max(acc, axis=0, keepdims=True)
        cpos = jnp.where(acc == vm, ci, BIG_I32)
        cls_ref[...] = jnp.concatenate(
            [jnp.min(cpos, axis=0, keepdims=True),
             jnp.zeros((7, q), jnp.int32)], axis=0)


def kernel(queries, keys, db_classes, k):
    del k
    q, d = queries.shape
    kvalid = keys.shape[0]
    nblocks = (kvalid + BLK - 1) // BLK
    kpad = nblocks * BLK
    keys_p = jnp.pad(keys, ((0, kpad - kvalid), (0, 0)))
    pay_full = (jnp.arange(kpad, dtype=jnp.int32) * 1024
                + jnp.pad(db_classes, (0, kpad - kvalid)))
    pay_p = jnp.broadcast_to(pay_full[:, None],
                             (kpad, 128)).reshape(nblocks, BLK, 128)

    out = pl.pallas_call(
        functools.partial(_knn_kernel, nblocks=nblocks, kvalid=kvalid, q=q),
        grid=(nblocks,),
        in_specs=[
            pl.BlockSpec((q, d), lambda j: (0, 0)),
            pl.BlockSpec((BLK, d), lambda j: (j, 0)),
            pl.BlockSpec((1, BLK, 128), lambda j: (j, 0, 0)),
        ],
        out_specs=[
            pl.BlockSpec((8, q), lambda j: (0, 0)),
            pl.BlockSpec((8, q), lambda j: (0, 0)),
            pl.BlockSpec((TOP_K, q), lambda j: (0, 0)),
        ],
        out_shape=[
            jax.ShapeDtypeStruct((8, q), jnp.float32),
            jax.ShapeDtypeStruct((8, q), jnp.int32),
            jax.ShapeDtypeStruct((TOP_K, q), jnp.float32),
        ],
        scratch_shapes=[
            pltpu.VMEM((q, d), jnp.float32),
            pltpu.VMEM((TOP_K, q), jnp.float32),
            pltpu.VMEM((TOP_K, q), jnp.int32),
            pltpu.VMEM((BLK, q), jnp.float32),
            pltpu.VMEM((BLK, q), jnp.int32),
            pltpu.SMEM((1,), jnp.int32),
        ],
        compiler_params=pltpu.CompilerParams(
            dimension_semantics=("arbitrary",)),
    )(queries, keys_p, pay_p)
    stats, clsout, topv_t = out
    return (stats[0], clsout[0], stats[1], jnp.transpose(topv_t))
